# Initial kernel scaffold; baseline (speedup 1.0000x reference)
import jax
import jax.numpy as jnp
from jax.experimental import pallas as pl


def kernel(x, emb, W, b):
    raise NotImplementedError("write your pallas kernel here")


# speedup vs baseline: 1.5551x; 1.5551x over previous
import jax
import jax.numpy as jnp
from jax import lax
from jax.experimental import pallas as pl
from jax.experimental.pallas import tpu as pltpu
from jax.experimental.pallas import tpu_sc as plsc

VOCAB = 36
POS = 7
ED = 32
OD = 128
B = 16384

NC = 2
NS = 16
NW = NC * NS
SPW = B // NW
NGRP = SPW // 16


def _table_body(emb_ref, w_ref, b_ref, t_ref):
    emb = emb_ref[...]
    for p in range(POS):
        tp = jnp.dot(emb, w_ref[p], preferred_element_type=jnp.float32)
        if p == POS - 1:
            tp = tp + b_ref[...]
        t_ref[p] = tp


def _build_table(emb, w3, b2):
    return pl.pallas_call(
        _table_body,
        out_shape=jax.ShapeDtypeStruct((POS, VOCAB, OD), jnp.float32),
    )(emb, w3, b2)


def _sc_body(t_hbm, x_hbm, out_hbm, tv, xv, ov):
    wid = lax.axis_index("s") * NC + lax.axis_index("c")
    base = wid * SPW
    pltpu.sync_copy(t_hbm, tv)
    pltpu.sync_copy(x_hbm.at[pl.ds(base * POS, SPW * POS)], xv)

    lanes = lax.iota(jnp.int32, 16)
    ovec = lanes * OD

    def group(g, _):
        rows7 = lanes * POS + g * (16 * POS)
        bases = []
        for p in range(POS):
            xi = plsc.load_gather(xv, [rows7 + p])
            bases.append(xi * OD + p * (VOCAB * OD))
        obase = ovec + g * (16 * OD)

        def cols(cg, _):
            for u in range(32):
                col = cg * 32 + u
                acc = plsc.load_gather(tv, [bases[0] + col])
                for p in range(1, POS):
                    acc = acc + plsc.load_gather(tv, [bases[p] + col])
                plsc.store_scatter(ov, [obase + col], acc)
            return 0

        lax.fori_loop(0, OD // 32, cols, 0)
        return 0

    lax.fori_loop(0, NGRP, group, 0)
    pltpu.sync_copy(ov, out_hbm.at[pl.ds(base * OD, SPW * OD)])


_sc_call = pl.kernel(
    _sc_body,
    mesh=plsc.VectorSubcoreMesh(core_axis_name="c", subcore_axis_name="s"),
    compiler_params=pltpu.CompilerParams(needs_layout_passes=False),
    out_type=jax.ShapeDtypeStruct((B * OD,), jnp.float32),
    scratch_types=[
        pltpu.VMEM((POS * VOCAB * OD,), jnp.float32),
        pltpu.VMEM((SPW * POS,), jnp.int32),
        pltpu.VMEM((SPW * OD,), jnp.float32),
    ],
)


def kernel(x, emb, W, b):
    t = _build_table(emb, W.reshape(POS, ED, OD), b.reshape(1, OD))
    out_flat = _sc_call(t.reshape(-1), x.reshape(-1).astype(jnp.int32))
    return out_flat.reshape(B, OD)


# speedup vs baseline: 5.6214x; 3.6147x over previous
import jax
import jax.numpy as jnp
from jax import lax
from jax.experimental import pallas as pl
from jax.experimental.pallas import tpu as pltpu
from jax.experimental.pallas import tpu_sc as plsc

VOCAB = 36
POS = 7
ED = 32
OD = 128
B = 16384

NC = 2
NS = 16
NW = NC * NS
SPW = B // NW


def _table_body(emb_ref, w_ref, b_ref, t_ref):
    emb = emb_ref[...]
    for p in range(POS):
        tp = jnp.dot(emb, w_ref[p], preferred_element_type=jnp.float32)
        if p == POS - 1:
            tp = tp + b_ref[...]
        t_ref[p] = tp


def _build_table(emb, w3, b2):
    return pl.pallas_call(
        _table_body,
        out_shape=jax.ShapeDtypeStruct((POS, VOCAB, OD), jnp.float32),
    )(emb, w3, b2)


def _sc_body(t_hbm, x_hbm, out_hbm, tv, xv, ov):
    wid = lax.axis_index("s") * NC + lax.axis_index("c")
    base = wid * SPW
    pltpu.sync_copy(t_hbm, tv)
    pltpu.sync_copy(x_hbm.at[pl.ds(base * 8, SPW * 8)], xv)

    def pair(s2, _):
        xvec = xv[pl.ds(s2 * 16, 16)]
        for h in range(2):
            addrs = [xvec[h * 8 + p] * OD + p * (VOCAB * OD) for p in range(POS)]
            obase = s2 * (2 * OD) + h * OD
            for j in range(OD // 16):
                acc = tv[pl.ds(addrs[0] + j * 16, 16)]
                for p in range(1, POS):
                    acc = acc + tv[pl.ds(addrs[p] + j * 16, 16)]
                ov[pl.ds(obase + j * 16, 16)] = acc
        return 0

    lax.fori_loop(0, SPW // 2, pair, 0)
    pltpu.sync_copy(ov, out_hbm.at[pl.ds(base * OD, SPW * OD)])


_sc_call = pl.kernel(
    _sc_body,
    mesh=plsc.VectorSubcoreMesh(core_axis_name="c", subcore_axis_name="s"),
    compiler_params=pltpu.CompilerParams(needs_layout_passes=False),
    out_type=jax.ShapeDtypeStruct((B * OD,), jnp.float32),
    scratch_types=[
        pltpu.VMEM((POS * VOCAB * OD,), jnp.float32),
        pltpu.VMEM((SPW * 8,), jnp.int32),
        pltpu.VMEM((SPW * OD,), jnp.float32),
    ],
)


def kernel(x, emb, W, b):
    t = _build_table(emb, W.reshape(POS, ED, OD), b.reshape(1, OD))
    x8 = jnp.pad(x.astype(jnp.int32), ((0, 0), (0, 1)))
    out_flat = _sc_call(t.reshape(-1), x8.reshape(-1))
    return out_flat.reshape(B, OD)


# speedup vs baseline: 8.1485x; 1.4496x over previous
import jax
import jax.numpy as jnp
from jax import lax
from jax.experimental import pallas as pl
from jax.experimental.pallas import tpu as pltpu
from jax.experimental.pallas import tpu_sc as plsc

VOCAB = 36
POS = 7
ED = 32
OD = 128
B = 16384

NC = 2
NS = 16
NW = NC * NS
SPW = B // NW


def _table_body(emb_ref, w_ref, b_ref, t_ref):
    emb = emb_ref[...]
    for p in range(POS):
        tp = jnp.dot(emb, w_ref[p], preferred_element_type=jnp.float32)
        if p == POS - 1:
            tp = tp + b_ref[...]
        t_ref[p] = tp


def _build_table(emb, w3, b2):
    return pl.pallas_call(
        _table_body,
        out_shape=jax.ShapeDtypeStruct((POS, VOCAB, OD), jnp.float32),
    )(emb, w3, b2)


def _sc_body(t_hbm, x_hbm, out_hbm, tv, xv, ov):
    wid = lax.axis_index("s") * NC + lax.axis_index("c")
    base = wid * SPW
    pltpu.sync_copy(t_hbm, tv)
    pltpu.sync_copy(x_hbm.at[pl.ds(base * 8, SPW * 8)], xv)

    lanes = lax.iota(jnp.int32, 16)
    cvec = (lanes % 8) * (VOCAB * OD)

    @plsc.parallel_loop(0, SPW // 2, 1, unroll=4)
    def pair(s2):
        xvec = xv[pl.ds(s2 * 16, 16)]
        av = xvec * OD + cvec
        for h in range(2):
            addrs = [av[h * 8 + p] for p in range(POS)]
            obase = s2 * (2 * OD) + h * OD
            for j in range(OD // 16):
                acc = tv[pl.ds(addrs[0] + j * 16, 16)]
                for p in range(1, POS):
                    acc = acc + tv[pl.ds(addrs[p] + j * 16, 16)]
                ov[pl.ds(obase + j * 16, 16)] = acc
    pltpu.sync_copy(ov, out_hbm.at[pl.ds(base * OD, SPW * OD)])


_sc_call = pl.kernel(
    _sc_body,
    mesh=plsc.VectorSubcoreMesh(core_axis_name="c", subcore_axis_name="s"),
    compiler_params=pltpu.CompilerParams(needs_layout_passes=False),
    out_type=jax.ShapeDtypeStruct((B * OD,), jnp.float32),
    scratch_types=[
        pltpu.VMEM((POS * VOCAB * OD,), jnp.float32),
        pltpu.VMEM((SPW * 8,), jnp.int32),
        pltpu.VMEM((SPW * OD,), jnp.float32),
    ],
)


def kernel(x, emb, W, b):
    t = _build_table(emb, W.reshape(POS, ED, OD), b.reshape(1, OD))
    x8 = jnp.pad(x.astype(jnp.int32), ((0, 0), (0, 1)))
    out_flat = _sc_call(t.reshape(-1), x8.reshape(-1))
    return out_flat.reshape(B, OD)
